# baseline (speedup 1.0000x reference)
import functools

import jax
import jax.numpy as jnp
from jax import lax
from jax.experimental import pallas as pl
from jax.experimental.pallas import tpu as pltpu
from jax.experimental.pallas import tpu_sc as plsc

_L = 16


def _sc_body(st_hbm, pos_hbm, out_hbm, buf, posv, sem):
    T, F, B = st_hbm.shape
    nw = 32
    w = lax.axis_index("s") * 2 + lax.axis_index("c")
    n_slabs = T * (F // 8)
    per_w = n_slabs // nw

    def do_slab(k, _):
        slab = w * per_w + k
        t = slab // (F // 8)
        i = slab % (F // 8)
        pltpu.async_copy(pos_hbm.at[T - 1 - t], posv, sem).wait()
        pltpu.async_copy(st_hbm.at[t, pl.ds(8 * i, 8)], buf, sem).wait()
        for r in range(8):
            f = 8 * i + r
            splat = plsc.load_gather(posv, [jnp.full((_L,), f, jnp.int32)])

            def add_vec(c, _, r=r, splat=splat):
                sl = pl.ds(c * _L, _L)
                buf[r, sl] = buf[r, sl] + splat
                return _

            lax.fori_loop(0, B // _L, add_vec, 0, unroll=8)
        pltpu.async_copy(buf, out_hbm.at[t, pl.ds(8 * i, 8)], sem).wait()
        return _

    lax.fori_loop(0, per_w, do_slab, 0)


def kernel(sessions, pos_emb):
    B, T, F = sessions.shape
    st = jnp.transpose(sessions, (1, 2, 0))
    mesh = plsc.VectorSubcoreMesh(core_axis_name="c", subcore_axis_name="s")
    out_t = pl.kernel(
        _sc_body,
        out_type=jax.ShapeDtypeStruct((T, F, B), jnp.float32),
        mesh=mesh,
        scratch_types=[
            pltpu.VMEM((8, B), jnp.float32),
            pltpu.VMEM((F,), jnp.float32),
            pltpu.SemaphoreType.DMA,
        ],
        compiler_params=pltpu.CompilerParams(
            use_tc_tiling_on_sc=True, needs_layout_passes=False
        ),
    )(st, pos_emb)
    return jnp.transpose(out_t, (2, 0, 1))


# speedup vs baseline: 1.7309x; 1.7309x over previous
import jax
import jax.numpy as jnp
from jax import lax
from jax.experimental import pallas as pl
from jax.experimental.pallas import tpu as pltpu
from jax.experimental.pallas import tpu_sc as plsc

_TB = 10
_ROWS_PER_WORKER = 7


def _sc_flip_body(pos_hbm, out_hbm, rowbuf, sem):
    T = pos_hbm.shape[0]
    w = lax.axis_index("s") * 2 + lax.axis_index("c")
    for k in range(_ROWS_PER_WORKER):
        r = w * _ROWS_PER_WORKER + k

        @pl.when(r < T)
        def _():
            pltpu.async_copy(pos_hbm.at[T - 1 - r], rowbuf, sem).wait()
            pltpu.async_copy(rowbuf, out_hbm.at[r], sem).wait()


def _tc_add_body(s_ref, p_ref, o_ref):
    jt = pl.program_id(0)
    for k in range(_TB):
        prow = p_ref[jt * _TB + k]
        o_ref[k] = s_ref[k] + jnp.broadcast_to(prow, s_ref.shape[1:])


def kernel(sessions, pos_emb):
    B, T, F = sessions.shape
    st = jnp.transpose(sessions, (1, 2, 0))

    mesh = plsc.VectorSubcoreMesh(core_axis_name="c", subcore_axis_name="s")
    pos_flipped = pl.kernel(
        _sc_flip_body,
        out_type=jax.ShapeDtypeStruct((T, F), jnp.float32),
        mesh=mesh,
        scratch_types=[
            pltpu.VMEM((F,), jnp.float32),
            pltpu.SemaphoreType.DMA,
        ],
        compiler_params=pltpu.CompilerParams(
            use_tc_tiling_on_sc=True, needs_layout_passes=False
        ),
    )(pos_emb)

    pos3 = pos_flipped[:, :, None]
    out_t = pl.pallas_call(
        _tc_add_body,
        grid=(T // _TB,),
        in_specs=[
            pl.BlockSpec((_TB, F, B), lambda jt: (jt, 0, 0)),
            pl.BlockSpec((T, F, 1), lambda jt: (0, 0, 0)),
        ],
        out_specs=pl.BlockSpec((_TB, F, B), lambda jt: (jt, 0, 0)),
        out_shape=jax.ShapeDtypeStruct((T, F, B), sessions.dtype),
        compiler_params=pltpu.CompilerParams(
            dimension_semantics=("arbitrary",),
            vmem_limit_bytes=100 * 1024 * 1024,
        ),
    )(st, pos3)
    return jnp.transpose(out_t, (2, 0, 1))


# speedup vs baseline: 1.7618x; 1.0179x over previous
import jax
import jax.numpy as jnp
from jax import lax
from jax.experimental import pallas as pl
from jax.experimental.pallas import tpu as pltpu
from jax.experimental.pallas import tpu_sc as plsc

_TB = 10
_ROWS_PER_WORKER = 7


def _sc_flip_body(pos_hbm, out_hbm, rowbuf, sem):
    T = pos_hbm.shape[0]
    w = lax.axis_index("s") * 2 + lax.axis_index("c")

    def each_row(fn):
        for k in range(_ROWS_PER_WORKER):
            r = w * _ROWS_PER_WORKER + k

            def _run(k=k, r=r):
                fn(k, r)

            pl.when(r < T)(_run)

    def fire_read(k, r):
        pltpu.async_copy(pos_hbm.at[T - 1 - r], rowbuf.at[k], sem)

    def drain_read(k, r):
        pltpu.make_async_copy(pos_hbm.at[T - 1 - r], rowbuf.at[k], sem).wait()

    def fire_write(k, r):
        pltpu.async_copy(rowbuf.at[k], out_hbm.at[r], sem)

    def drain_write(k, r):
        pltpu.make_async_copy(rowbuf.at[k], out_hbm.at[r], sem).wait()

    each_row(fire_read)
    each_row(drain_read)
    each_row(fire_write)
    each_row(drain_write)


def _tc_add_body(s_ref, p_ref, o_ref):
    jt = pl.program_id(0)
    for k in range(_TB):
        prow = p_ref[jt * _TB + k]
        o_ref[k] = s_ref[k] + jnp.broadcast_to(prow, s_ref.shape[1:])


def kernel(sessions, pos_emb):
    B, T, F = sessions.shape
    st = jnp.transpose(sessions, (1, 2, 0))

    mesh = plsc.VectorSubcoreMesh(core_axis_name="c", subcore_axis_name="s")
    pos_flipped = pl.kernel(
        _sc_flip_body,
        out_type=jax.ShapeDtypeStruct((T, F), jnp.float32),
        mesh=mesh,
        scratch_types=[
            pltpu.VMEM((_ROWS_PER_WORKER, F), jnp.float32),
            pltpu.SemaphoreType.DMA,
        ],
        compiler_params=pltpu.CompilerParams(
            use_tc_tiling_on_sc=True, needs_layout_passes=False
        ),
    )(pos_emb)

    pos3 = pos_flipped[:, :, None]
    out_t = pl.pallas_call(
        _tc_add_body,
        grid=(T // _TB,),
        in_specs=[
            pl.BlockSpec((_TB, F, B), lambda jt: (jt, 0, 0)),
            pl.BlockSpec((T, F, 1), lambda jt: (0, 0, 0)),
        ],
        out_specs=pl.BlockSpec((_TB, F, B), lambda jt: (jt, 0, 0)),
        out_shape=jax.ShapeDtypeStruct((T, F, B), sessions.dtype),
        compiler_params=pltpu.CompilerParams(
            dimension_semantics=("arbitrary",),
            vmem_limit_bytes=100 * 1024 * 1024,
        ),
    )(st, pos3)
    return jnp.transpose(out_t, (2, 0, 1))
